# Initial kernel scaffold; baseline (speedup 1.0000x reference)
import jax
import jax.numpy as jnp
from jax.experimental import pallas as pl


def kernel(pos_u, pos_v, neg_v, u_weight, v_weight):
    raise NotImplementedError("write your pallas kernel here")


# speedup vs baseline: 4.6961x; 4.6961x over previous
import functools

import jax
import jax.numpy as jnp
from jax import lax
from jax.experimental import pallas as pl
from jax.experimental.pallas import tpu as pltpu
from jax.experimental.pallas import tpu_sc as plsc

NC = 2
NS = 16
NW = NC * NS


CH = 256


def _sc_gather(u_weight, v_weight, pos_u, idx_v6, B, D):
    bpw = B // NW
    nu = bpw // CH
    nv = 6 * bpw // CH
    nt = nu + nv
    pos_u = pos_u.reshape(NW, nu, CH)
    idx_v6 = idx_v6.reshape(NW, nv, CH)

    mesh = plsc.VectorSubcoreMesh(core_axis_name="c", subcore_axis_name="s")

    @functools.partial(
        pl.kernel,
        mesh=mesh,
        compiler_params=pltpu.CompilerParams(use_tc_tiling_on_sc=False),
        out_type=[
            jax.ShapeDtypeStruct((B, D), jnp.float32),
            jax.ShapeDtypeStruct((6 * B, D), jnp.float32),
        ],
        scratch_types=[
            pltpu.VMEM((nu, CH), jnp.int32),
            pltpu.VMEM((nv, CH), jnp.int32),
            pltpu.VMEM((CH, D), jnp.float32),
            pltpu.VMEM((CH, D), jnp.float32),
            pltpu.SemaphoreType.DMA,
            pltpu.SemaphoreType.DMA,
        ],
    )
    def k(uw_hbm, vw_hbm, idxu_hbm, idxv_hbm, outu_hbm, outv_hbm,
          idxu_v, idxv_v, rows_a, rows_b, sem_a, sem_b):
        wid = lax.axis_index("s") * NC + lax.axis_index("c")
        ubase = wid * bpw
        vbase = wid * 6 * bpw
        pltpu.sync_copy(idxu_hbm.at[wid], idxu_v)
        pltpu.sync_copy(idxv_hbm.at[wid], idxv_v)

        def gather_of(t, buf, sem):
            if t < nu:
                return pltpu.async_copy(uw_hbm.at[idxu_v.at[t]], buf, sem)
            return pltpu.async_copy(vw_hbm.at[idxv_v.at[t - nu]], buf, sem)

        def writeback(t, buf):
            if t < nu:
                pltpu.sync_copy(buf, outu_hbm.at[pl.ds(ubase + t * CH, CH)])
            else:
                pltpu.sync_copy(
                    buf, outv_hbm.at[pl.ds(vbase + (t - nu) * CH, CH)])

        copies = {
            0: gather_of(0, rows_a, sem_a),
            1: gather_of(1, rows_b, sem_b),
        }
        for t in range(nt):
            buf, sem = (rows_a, sem_a) if t % 2 == 0 else (rows_b, sem_b)
            copies[t].wait()
            writeback(t, buf)
            if t + 2 < nt:
                copies[t + 2] = gather_of(t + 2, buf, sem)

    return k(u_weight, v_weight, pos_u, idx_v6)


def _tc_loss(emb_u, emb_v6, B, D, nb):
    def body(u_ref, v6_ref, out_ref):
        i = pl.program_id(0)
        u = u_ref[...]
        s = jnp.sum(u * v6_ref[0], axis=1)
        s = jnp.clip(s, -10.0, 10.0)
        loss = jnp.log1p(jnp.exp(-s))
        for k in range(1, 6):
            t = jnp.sum(u * v6_ref[k], axis=1)
            t = jnp.clip(t, -10.0, 10.0)
            loss = loss + jnp.log1p(jnp.exp(t))
        total = jnp.sum(loss)

        @pl.when(i == 0)
        def _():
            out_ref[...] = jnp.zeros_like(out_ref)

        out_ref[...] = out_ref[...] + total

    out = pl.pallas_call(
        body,
        grid=(B // nb,),
        in_specs=[
            pl.BlockSpec((nb, D), lambda i: (i, 0)),
            pl.BlockSpec((6, nb, D), lambda i: (0, i, 0)),
        ],
        out_specs=pl.BlockSpec((1, 1), lambda i: (0, 0)),
        out_shape=jax.ShapeDtypeStruct((1, 1), jnp.float32),
    )(emb_u, emb_v6)
    return out[0, 0]


def kernel(pos_u, pos_v, neg_v, u_weight, v_weight):
    B = pos_u.shape[0]
    D = u_weight.shape[1]
    neg_flat = jnp.swapaxes(neg_v, 0, 1).reshape(-1)
    idx_v6 = jnp.concatenate([pos_v, neg_flat])

    emb_u, emb_v6 = _sc_gather(u_weight, v_weight, pos_u, idx_v6, B, D)
    total = _tc_loss(emb_u, emb_v6.reshape(6, B, D), B, D, nb=2048)
    return total / B
